# baseline (speedup 1.0000x reference)
import functools
import math

import jax
import jax.numpy as jnp
from jax import lax
from jax.experimental import pallas as pl
from jax.experimental.pallas import tpu as pltpu
from jax.experimental.pallas import tpu_sc as plsc

D_MODEL = 128
SCALE = math.sqrt(D_MODEL)
NC, NS, LANES = 2, 16, 16
NW = NC * NS
G = 4
SPLITS = 4
ROWS_PER_BLOCK = 8


def _gather_body(x_hbm, table_hbm, out_hbm, idx_v, gin, gsem, ssem):
    n_rows = x_hbm.shape[0]
    seq = x_hbm.shape[1]
    rows_per_w = n_rows // NW
    n_steps = rows_per_w // G
    wid = lax.axis_index("s") * NC + lax.axis_index("c")
    base = wid * rows_per_w
    pltpu.sync_copy(x_hbm.at[pl.ds(base, rows_per_w)], idx_v)

    def start_gathers(step, b):
        for g in range(G):
            pltpu.async_copy(table_hbm.at[idx_v.at[step * G + g]],
                             gin.at[b, pl.ds(g * seq, seq)], gsem.at[b])

    def wait_gathers(step, b):
        for g in range(G):
            pltpu.make_async_copy(table_hbm.at[idx_v.at[step * G + g]],
                                  gin.at[b, pl.ds(g * seq, seq)],
                                  gsem.at[b]).wait()

    def wait_scatter(b):
        pltpu.make_async_copy(gin.at[b], out_hbm.at[pl.ds(0, G * seq)],
                              ssem.at[b]).wait()

    start_gathers(0, 0)

    def outer(p, carry):
        for b in range(2):
            step_ref = 2 * p + b
            wait_gathers(step_ref, b)
            pltpu.async_copy(
                gin.at[b],
                out_hbm.at[pl.ds((base + step_ref * G) * seq, G * seq)],
                ssem.at[b])
            if b == 0:
                @pl.when(p >= 1)
                def _():
                    wait_scatter(1)
                start_gathers(step_ref + 1, 1)
            else:
                @pl.when(p < n_steps // 2 - 1)
                def _():
                    wait_scatter(0)
                    start_gathers(step_ref + 1, 0)
        return carry

    lax.fori_loop(0, n_steps // 2, outer, 0)
    wait_scatter(0)
    wait_scatter(1)


def _sc_gather(x_part, table):
    n_rows, seq = x_part.shape
    mesh = plsc.VectorSubcoreMesh(core_axis_name="c", subcore_axis_name="s")
    return pl.kernel(
        _gather_body,
        out_type=jax.ShapeDtypeStruct((n_rows * seq, D_MODEL), jnp.float32),
        mesh=mesh,
        compiler_params=pltpu.CompilerParams(use_tc_tiling_on_sc=True),
        scratch_types=[
            pltpu.VMEM((n_rows // NW, seq), jnp.int32),
            pltpu.VMEM((2, G * seq, D_MODEL), jnp.float32),
            pltpu.SemaphoreType.DMA((2,)),
            pltpu.SemaphoreType.DMA((2,)),
        ],
    )(x_part, table)


def _scale_kernel(flat_ref, o_ref):
    o_ref[...] = flat_ref[...].reshape(o_ref.shape) * SCALE


def _scale_kernel_acc(acc_ref, flat_ref, o_ref):
    del acc_ref
    o_ref[...] = flat_ref[...].reshape(o_ref.shape) * SCALE


def _tc_finish(flat, acc, s, part, n_rows, seq):
    grid = (part // ROWS_PER_BLOCK,)
    blk = ROWS_PER_BLOCK
    off = s * part // blk
    in_spec = pl.BlockSpec((blk * seq, D_MODEL), lambda i: (i, 0))
    out_spec = pl.BlockSpec((blk, seq, D_MODEL),
                            lambda i, off=off: (i + off, 0, 0))
    out_shape = jax.ShapeDtypeStruct((n_rows, seq, D_MODEL), jnp.float32)
    if acc is None:
        return pl.pallas_call(
            _scale_kernel, grid=grid, in_specs=[in_spec],
            out_specs=out_spec, out_shape=out_shape)(flat)
    acc_spec = pl.BlockSpec(memory_space=pl.ANY)
    return pl.pallas_call(
        _scale_kernel_acc, grid=grid, in_specs=[acc_spec, in_spec],
        out_specs=out_spec, out_shape=out_shape,
        input_output_aliases={0: 0})(acc, flat)


def kernel(x, table):
    n_rows, seq = x.shape
    part = n_rows // SPLITS
    flats = [_sc_gather(x[s * part:(s + 1) * part], table)
             for s in range(SPLITS)]
    acc = None
    for s in range(SPLITS):
        acc = _tc_finish(flats[s], acc, s, part, n_rows, seq)
    return acc
